# baseline (speedup 1.0000x reference)
import functools

import jax
import jax.numpy as jnp
from jax import lax
from jax.experimental import pallas as pl
from jax.experimental.pallas import tpu as pltpu
from jax.experimental.pallas import tpu_sc as plsc


_ROWS_BLK = 8000


def _double_body(t_ref, o_ref):
    t = t_ref[...]
    o_ref[...] = t + t


def _double(tensor):
    m, d = tensor.shape
    blk = _ROWS_BLK if m % _ROWS_BLK == 0 else m
    grid = m // blk
    return pl.pallas_call(
        _double_body,
        grid=(grid,),
        in_specs=[pl.BlockSpec((blk, d), lambda i: (i, 0))],
        out_specs=pl.BlockSpec((blk, d), lambda i: (i, 0)),
        out_shape=jax.ShapeDtypeStruct((m, d), tensor.dtype),
        compiler_params=pltpu.CompilerParams(
            dimension_semantics=("arbitrary",)),
    )(tensor)


_JB = 512


def _delta_body(idx_col_ref, idx_row_ref, val_ref, o_ref):
    eq = idx_col_ref[...] == idx_row_ref[...]
    e = jnp.where(eq, jnp.float32(2.0), jnp.float32(0.0))
    o_ref[...] = lax.dot(
        e, val_ref[...],
        precision=lax.Precision.HIGHEST,
        preferred_element_type=jnp.float32,
    )


def _delta(indices, val):
    b, d = val.shape
    jb = _JB if b % _JB == 0 else b
    grid = b // jb
    idx_col = indices.reshape(b, 1)
    idx_row = indices.reshape(1, b)
    return pl.pallas_call(
        _delta_body,
        grid=(grid,),
        in_specs=[
            pl.BlockSpec((jb, 1), lambda i: (i, 0)),
            pl.BlockSpec((1, b), lambda i: (0, 0)),
            pl.BlockSpec((b, d), lambda i: (0, 0)),
        ],
        out_specs=pl.BlockSpec((jb, d), lambda i: (i, 0)),
        out_shape=jax.ShapeDtypeStruct((b, d), jnp.float32),
        compiler_params=pltpu.CompilerParams(
            dimension_semantics=("arbitrary",)),
    )(idx_col, idx_row, val)


_NSUB = 16
_IDXW = 128


def _sc_scatter_body(out_ref, delta_hbm, idx_hbm, idx_v, rows_a, rows_b,
                     delta_v, sem):
    c = lax.axis_index("c")
    s = lax.axis_index("s")

    @pl.when(c == 0)
    def _():
        pltpu.sync_copy(idx_hbm.at[pl.ds(2 * s, 2)], idx_v)
        cp0 = pltpu.async_copy(out_ref.at[idx_v.at[0]], rows_a, sem)
        cp1 = pltpu.async_copy(out_ref.at[idx_v.at[1]], rows_b, sem)
        pltpu.sync_copy(delta_hbm.at[pl.ds(s * 2 * _IDXW, 2 * _IDXW)], delta_v)
        cp0.wait()
        cp1.wait()

        def add_row(r, _):
            for half, rows in ((0, rows_a), (1, rows_b)):
                for col in range(4):
                    sl = pl.ds(col * 16, 16)
                    rows[r, sl] = rows[r, sl] + delta_v[half * _IDXW + r, sl]
            return 0

        lax.fori_loop(0, _IDXW, add_row, 0)

        plsc.subcore_barrier()

        cp2 = pltpu.async_copy(rows_a, out_ref.at[idx_v.at[0]], sem)
        cp3 = pltpu.async_copy(rows_b, out_ref.at[idx_v.at[1]], sem)
        cp2.wait()
        cp3.wait()


def _sc_scatter(out1_ref, delta, indices):
    b = indices.shape[0]
    d = delta.shape[1]
    idx2d = indices.reshape(b // _IDXW, _IDXW)
    mesh = plsc.VectorSubcoreMesh(
        core_axis_name="c", subcore_axis_name="s", num_cores=2, num_subcores=16)
    run = pl.kernel(
        _sc_scatter_body,
        out_type=(),
        mesh=mesh,
        scratch_types=[
            pltpu.VMEM((2, _IDXW), jnp.int32),
            pltpu.VMEM((_IDXW, d), jnp.float32),
            pltpu.VMEM((_IDXW, d), jnp.float32),
            pltpu.VMEM((2 * _IDXW, d), jnp.float32),
            pltpu.SemaphoreType.DMA,
        ],
        compiler_params=pltpu.CompilerParams(use_tc_tiling_on_sc=False),
    )
    run(out1_ref, delta, idx2d)


def kernel(tensor, val, indices):
    out1 = _double(tensor)
    delta = _delta(indices, val)
    ref = jax.new_ref(out1)
    _sc_scatter(ref, delta, indices)
    return jax.freeze(ref)
